# baseline (speedup 1.0000x reference)
import functools

import jax
import jax.numpy as jnp
from jax import lax
from jax.experimental import pallas as pl
from jax.experimental.pallas import tpu as pltpu
from jax.experimental.pallas import tpu_sc as plsc

_N_NODES = 10000
_N_EDGES = 320000
_D = 128
_N_GRAPHS = 64

_NC = 2
_NS = 16
_NW = _NC * _NS
_CHUNK = 88
_NCHUNK = 116
_EPW = _NCHUNK * _CHUNK
_NBUF = 4
_NSUPER = _NCHUNK // _NBUF
_ZROWS = 640
_ACC_ROWS = _NS * _ZROWS
_PAD_DST = _ACC_ROWS - 1


def _edge_agg_body(h_hbm, src_hbm, dst_hbm, zeros_hbm, out_hbm,
                   sidx, didx, rows, acc, isems, gsems, ssems):
    c = lax.axis_index("c")
    s = lax.axis_index("s")
    wid = c * _NS + s

    zb = s * _ZROWS
    pltpu.sync_copy(zeros_hbm, acc.at[pl.ds(zb, _ZROWS)])
    plsc.subcore_barrier()

    def issue_idx(g, b):
        pltpu.async_copy(src_hbm.at[wid, g], sidx[b], isems[b])
        pltpu.async_copy(dst_hbm.at[wid, g], didx[b], isems[b])

    def wait_idx(g, b):
        pltpu.make_async_copy(src_hbm.at[wid, g], sidx[b], isems[b]).wait()
        pltpu.make_async_copy(dst_hbm.at[wid, g], didx[b], isems[b]).wait()

    def issue_gather(b):
        pltpu.async_copy(h_hbm.at[sidx[b]], rows[b], gsems[b])

    def wait_gather(b):
        pltpu.make_async_copy(h_hbm.at[sidx[b]], rows[b], gsems[b]).wait()

    def scatter(b):
        pltpu.sync_copy(rows[b], acc.at[didx[b]], add=True)

    for b in range(_NBUF):
        issue_idx(b, b)

    def super_body(i, carry):
        g0 = i * _NBUF
        for b in range(_NBUF):
            wait_idx(g0 + b, b)
            issue_gather(b)
        for b in range(_NBUF):
            wait_gather(b)
            scatter(b)
            issue_idx(g0 + _NBUF + b, b)
        return carry

    lax.fori_loop(0, _NCHUNK // _NBUF - 1, super_body, 0)

    g0 = _NCHUNK - _NBUF
    for b in range(_NBUF):
        wait_idx(g0 + b, b)
        issue_gather(b)
    for b in range(_NBUF):
        wait_gather(b)
        scatter(b)

    plsc.subcore_barrier()
    pltpu.sync_copy(acc.at[pl.ds(zb, _ZROWS)], out_hbm.at[c, pl.ds(zb, _ZROWS)])


_edge_agg = functools.partial(
    pl.kernel,
    out_type=jax.ShapeDtypeStruct((_NC, _ACC_ROWS, _D), jnp.float32),
    mesh=plsc.VectorSubcoreMesh(core_axis_name="c", subcore_axis_name="s"),
    scratch_types=[
        [pltpu.VMEM((_CHUNK,), jnp.int32) for _ in range(_NBUF)],
        [pltpu.VMEM((_CHUNK,), jnp.int32) for _ in range(_NBUF)],
        [pltpu.VMEM((_CHUNK, _D), jnp.float32) for _ in range(_NBUF)],
        pltpu.VMEM_SHARED((_ACC_ROWS, _D), jnp.float32),
        [pltpu.SemaphoreType.DMA for _ in range(_NBUF)],
        [pltpu.SemaphoreType.DMA for _ in range(_NBUF)],
        [pltpu.SemaphoreType.DMA for _ in range(_NBUF)],
    ],
)(_edge_agg_body)


def _mlp_bn_kernel(p_ref, w1_ref, b1_ref, w2_ref, b2_ref, g_ref, be_ref,
                   out_ref):
    pooled = p_ref[0, :_N_NODES, :] + p_ref[1, :_N_NODES, :]
    h = jnp.dot(pooled, w1_ref[...], preferred_element_type=jnp.float32)
    h = jnp.maximum(h + b1_ref[...], 0.0)
    z = jnp.dot(h, w2_ref[...], preferred_element_type=jnp.float32)
    z = z + b2_ref[...]
    mean = jnp.mean(z, axis=0, keepdims=True)
    var = jnp.mean((z - mean) * (z - mean), axis=0, keepdims=True)
    hn = (z - mean) * lax.rsqrt(var + 1e-5) * g_ref[...] + be_ref[...]
    out_ref[...] = jnp.maximum(hn, 0.0)


def _mlp_bn_pool_kernel(p_ref, gid_ref, w1_ref, b1_ref, w2_ref, b2_ref,
                        g_ref, be_ref, out_h_ref, out_p_ref):
    pooled = p_ref[0, :_N_NODES, :] + p_ref[1, :_N_NODES, :]
    h = jnp.dot(pooled, w1_ref[...], preferred_element_type=jnp.float32)
    h = jnp.maximum(h + b1_ref[...], 0.0)
    z = jnp.dot(h, w2_ref[...], preferred_element_type=jnp.float32)
    z = z + b2_ref[...]
    mean = jnp.mean(z, axis=0, keepdims=True)
    var = jnp.mean((z - mean) * (z - mean), axis=0, keepdims=True)
    hn = (z - mean) * lax.rsqrt(var + 1e-5) * g_ref[...] + be_ref[...]
    hr = jnp.maximum(hn, 0.0)
    out_h_ref[...] = hr
    oh = (lax.broadcasted_iota(jnp.int32, (_N_GRAPHS, _N_NODES), 0)
          == gid_ref[...]).astype(jnp.float32)
    out_p_ref[...] = jnp.dot(oh, hr, preferred_element_type=jnp.float32)


def _mlp_bn(p, w1, b1, w2, b2, gamma, beta):
    return pl.pallas_call(
        _mlp_bn_kernel,
        out_shape=jax.ShapeDtypeStruct((_N_NODES, _D), jnp.float32),
    )(p, w1, b1.reshape(1, _D), w2, b2.reshape(1, _D),
      gamma.reshape(1, _D), beta.reshape(1, _D))


def _mlp_bn_pool(p, gids, w1, b1, w2, b2, gamma, beta):
    return pl.pallas_call(
        _mlp_bn_pool_kernel,
        out_shape=(jax.ShapeDtypeStruct((_N_NODES, _D), jnp.float32),
                   jax.ShapeDtypeStruct((_N_GRAPHS, _D), jnp.float32)),
    )(p, gids.reshape(1, _N_NODES), w1, b1.reshape(1, _D), w2,
      b2.reshape(1, _D), gamma.reshape(1, _D), beta.reshape(1, _D))


def kernel(x, edge_index, graph_ids,
           W1_0, b1_0, W2_0, b2_0, gamma_0, beta_0,
           W1_1, b1_1, W2_1, b2_1, gamma_1, beta_1):
    pad = _NW * _EPW - _N_EDGES
    pad_iota = jnp.arange(pad, dtype=jnp.int32)
    src = jnp.concatenate(
        [edge_index[0], pad_iota % _N_NODES]
    ).reshape(_NW, _NCHUNK, _CHUNK)
    n_junk = _ACC_ROWS - _N_NODES
    dst = jnp.concatenate(
        [edge_index[1], _N_NODES + (pad_iota % n_junk)]
    ).reshape(_NW, _NCHUNK, _CHUNK)
    zeros = jnp.zeros((_ZROWS, _D), jnp.float32)

    p0 = _edge_agg(x, src, dst, zeros)
    h1 = _mlp_bn(p0, W1_0, b1_0, W2_0, b2_0, gamma_0, beta_0)
    p1 = _edge_agg(h1, src, dst, zeros)
    h2, pooled_x = _mlp_bn_pool(p1, graph_ids, W1_1, b1_1, W2_1, b2_1,
                                gamma_1, beta_1)
    return (pooled_x, h2)


# speedup vs baseline: 1.0244x; 1.0244x over previous
import functools

import jax
import jax.numpy as jnp
from jax import lax
from jax.experimental import pallas as pl
from jax.experimental.pallas import tpu as pltpu
from jax.experimental.pallas import tpu_sc as plsc

_N_NODES = 10000
_N_EDGES = 320000
_D = 128
_N_GRAPHS = 64

_NC = 2
_NS = 16
_NW = _NC * _NS
_CHUNK = 120
_NCHUNK = 84
_EPW = _NCHUNK * _CHUNK
_NBUF = 2
_NSUPER = _NCHUNK // _NBUF
_ZROWS = 640
_ACC_ROWS = _NS * _ZROWS
_PAD_DST = _ACC_ROWS - 1


def _edge_agg_body(h_hbm, src_hbm, dst_hbm, zeros_hbm, out_hbm,
                   sidx, didx, rows, acc, isems, gsems, ssems):
    c = lax.axis_index("c")
    s = lax.axis_index("s")
    wid = c * _NS + s

    zb = s * _ZROWS
    pltpu.sync_copy(zeros_hbm, acc.at[pl.ds(zb, _ZROWS)])
    plsc.subcore_barrier()

    def issue_idx(g, b):
        pltpu.async_copy(src_hbm.at[wid, g], sidx[b], isems[b])
        pltpu.async_copy(dst_hbm.at[wid, g], didx[b], isems[b])

    def wait_idx(g, b):
        pltpu.make_async_copy(src_hbm.at[wid, g], sidx[b], isems[b]).wait()
        pltpu.make_async_copy(dst_hbm.at[wid, g], didx[b], isems[b]).wait()

    def issue_gather(b):
        pltpu.async_copy(h_hbm.at[sidx[b]], rows[b], gsems[b])

    def wait_gather(b):
        pltpu.make_async_copy(h_hbm.at[sidx[b]], rows[b], gsems[b]).wait()

    def scatter(b):
        pltpu.sync_copy(rows[b], acc.at[didx[b]], add=True)

    for b in range(_NBUF):
        issue_idx(b, b)

    def super_body(i, carry):
        g0 = i * _NBUF
        for b in range(_NBUF):
            wait_idx(g0 + b, b)
            issue_gather(b)
        for b in range(_NBUF):
            wait_gather(b)
            scatter(b)
            issue_idx(g0 + _NBUF + b, b)
        return carry

    lax.fori_loop(0, _NCHUNK // _NBUF - 1, super_body, 0)

    g0 = _NCHUNK - _NBUF
    for b in range(_NBUF):
        wait_idx(g0 + b, b)
        issue_gather(b)
    for b in range(_NBUF):
        wait_gather(b)
        scatter(b)

    plsc.subcore_barrier()
    pltpu.sync_copy(acc.at[pl.ds(zb, _ZROWS)], out_hbm.at[c, pl.ds(zb, _ZROWS)])


_edge_agg = functools.partial(
    pl.kernel,
    out_type=jax.ShapeDtypeStruct((_NC, _ACC_ROWS, _D), jnp.float32),
    mesh=plsc.VectorSubcoreMesh(core_axis_name="c", subcore_axis_name="s"),
    scratch_types=[
        [pltpu.VMEM((_CHUNK,), jnp.int32) for _ in range(_NBUF)],
        [pltpu.VMEM((_CHUNK,), jnp.int32) for _ in range(_NBUF)],
        [pltpu.VMEM((_CHUNK, _D), jnp.float32) for _ in range(_NBUF)],
        pltpu.VMEM_SHARED((_ACC_ROWS, _D), jnp.float32),
        [pltpu.SemaphoreType.DMA for _ in range(_NBUF)],
        [pltpu.SemaphoreType.DMA for _ in range(_NBUF)],
        [pltpu.SemaphoreType.DMA for _ in range(_NBUF)],
    ],
)(_edge_agg_body)


def _mlp_bn_kernel(p_ref, w1_ref, b1_ref, w2_ref, b2_ref, g_ref, be_ref,
                   out_ref):
    pooled = p_ref[0, :_N_NODES, :] + p_ref[1, :_N_NODES, :]
    h = jnp.dot(pooled, w1_ref[...], preferred_element_type=jnp.float32)
    h = jnp.maximum(h + b1_ref[...], 0.0)
    z = jnp.dot(h, w2_ref[...], preferred_element_type=jnp.float32)
    z = z + b2_ref[...]
    mean = jnp.mean(z, axis=0, keepdims=True)
    var = jnp.mean((z - mean) * (z - mean), axis=0, keepdims=True)
    hn = (z - mean) * lax.rsqrt(var + 1e-5) * g_ref[...] + be_ref[...]
    out_ref[...] = jnp.maximum(hn, 0.0)


def _mlp_bn_pool_kernel(p_ref, gid_ref, w1_ref, b1_ref, w2_ref, b2_ref,
                        g_ref, be_ref, out_h_ref, out_p_ref):
    pooled = p_ref[0, :_N_NODES, :] + p_ref[1, :_N_NODES, :]
    h = jnp.dot(pooled, w1_ref[...], preferred_element_type=jnp.float32)
    h = jnp.maximum(h + b1_ref[...], 0.0)
    z = jnp.dot(h, w2_ref[...], preferred_element_type=jnp.float32)
    z = z + b2_ref[...]
    mean = jnp.mean(z, axis=0, keepdims=True)
    var = jnp.mean((z - mean) * (z - mean), axis=0, keepdims=True)
    hn = (z - mean) * lax.rsqrt(var + 1e-5) * g_ref[...] + be_ref[...]
    hr = jnp.maximum(hn, 0.0)
    out_h_ref[...] = hr
    oh = (lax.broadcasted_iota(jnp.int32, (_N_GRAPHS, _N_NODES), 0)
          == gid_ref[...]).astype(jnp.float32)
    out_p_ref[...] = jnp.dot(oh, hr, preferred_element_type=jnp.float32)


def _mlp_bn(p, w1, b1, w2, b2, gamma, beta):
    return pl.pallas_call(
        _mlp_bn_kernel,
        out_shape=jax.ShapeDtypeStruct((_N_NODES, _D), jnp.float32),
    )(p, w1, b1.reshape(1, _D), w2, b2.reshape(1, _D),
      gamma.reshape(1, _D), beta.reshape(1, _D))


def _mlp_bn_pool(p, gids, w1, b1, w2, b2, gamma, beta):
    return pl.pallas_call(
        _mlp_bn_pool_kernel,
        out_shape=(jax.ShapeDtypeStruct((_N_NODES, _D), jnp.float32),
                   jax.ShapeDtypeStruct((_N_GRAPHS, _D), jnp.float32)),
    )(p, gids.reshape(1, _N_NODES), w1, b1.reshape(1, _D), w2,
      b2.reshape(1, _D), gamma.reshape(1, _D), beta.reshape(1, _D))


def kernel(x, edge_index, graph_ids,
           W1_0, b1_0, W2_0, b2_0, gamma_0, beta_0,
           W1_1, b1_1, W2_1, b2_1, gamma_1, beta_1):
    pad = _NW * _EPW - _N_EDGES
    pad_iota = jnp.arange(pad, dtype=jnp.int32)
    src = jnp.concatenate(
        [edge_index[0], pad_iota % _N_NODES]
    ).reshape(_NW, _NCHUNK, _CHUNK)
    n_junk = _ACC_ROWS - _N_NODES
    dst = jnp.concatenate(
        [edge_index[1], _N_NODES + (pad_iota % n_junk)]
    ).reshape(_NW, _NCHUNK, _CHUNK)
    zeros = jnp.zeros((_ZROWS, _D), jnp.float32)

    p0 = _edge_agg(x, src, dst, zeros)
    h1 = _mlp_bn(p0, W1_0, b1_0, W2_0, b2_0, gamma_0, beta_0)
    p1 = _edge_agg(h1, src, dst, zeros)
    h2, pooled_x = _mlp_bn_pool(p1, graph_ids, W1_1, b1_1, W2_1, b2_1,
                                gamma_1, beta_1)
    return (pooled_x, h2)


# speedup vs baseline: 1.0372x; 1.0125x over previous
import functools

import jax
import jax.numpy as jnp
from jax import lax
from jax.experimental import pallas as pl
from jax.experimental.pallas import tpu as pltpu
from jax.experimental.pallas import tpu_sc as plsc

_N_NODES = 10000
_N_EDGES = 320000
_D = 128
_N_GRAPHS = 64

_NC = 2
_NS = 16
_NW = _NC * _NS
_CHUNK = 120
_NCHUNK = 84
_EPW = _NCHUNK * _CHUNK
_NBUF = 2
_NSUPER = _NCHUNK // _NBUF
_ZROWS = 640
_ACC_ROWS = _NS * _ZROWS
_PAD_DST = _ACC_ROWS - 1


def _edge_agg_body(h_hbm, src_hbm, dst_hbm, out_hbm,
                   sidx, didx, rows, acc, isems, gsems):
    c = lax.axis_index("c")
    s = lax.axis_index("s")
    wid = c * _NS + s
    zb = s * _ZROWS

    def issue_idx(g, b):
        pltpu.async_copy(src_hbm.at[wid, g], sidx[b], isems[b])
        pltpu.async_copy(dst_hbm.at[wid, g], didx[b], isems[b])

    def wait_idx(g, b):
        pltpu.make_async_copy(src_hbm.at[wid, g], sidx[b], isems[b]).wait()
        pltpu.make_async_copy(dst_hbm.at[wid, g], didx[b], isems[b]).wait()

    def issue_gather(b):
        pltpu.async_copy(h_hbm.at[sidx[b]], rows[b], gsems[b])

    def wait_gather(b):
        pltpu.make_async_copy(h_hbm.at[sidx[b]], rows[b], gsems[b]).wait()

    def scatter(b):
        pltpu.sync_copy(rows[b], acc.at[didx[b]], add=True)

    for b in range(_NBUF):
        issue_idx(b, b)

    zvec = jnp.zeros((16,), jnp.float32)

    def zstore(i, carry):
        rows[0][i // 8, pl.ds((i % 8) * 16, 16)] = zvec
        return carry

    lax.fori_loop(0, _CHUNK * (_D // 16), zstore, 0)
    for k in range(_ZROWS // _CHUNK):
        pltpu.sync_copy(rows[0], acc.at[pl.ds(zb + k * _CHUNK, _CHUNK)])
    _REM = _ZROWS % _CHUNK
    if _REM:
        pltpu.sync_copy(rows[0].at[pl.ds(0, _REM)],
                        acc.at[pl.ds(zb + (_ZROWS // _CHUNK) * _CHUNK, _REM)])
    plsc.subcore_barrier()

    def super_body(i, carry):
        g0 = i * _NBUF
        for b in range(_NBUF):
            wait_idx(g0 + b, b)
            issue_gather(b)
        for b in range(_NBUF):
            wait_gather(b)
            scatter(b)
            issue_idx(g0 + _NBUF + b, b)
        return carry

    lax.fori_loop(0, _NCHUNK // _NBUF - 1, super_body, 0)

    g0 = _NCHUNK - _NBUF
    for b in range(_NBUF):
        wait_idx(g0 + b, b)
        issue_gather(b)
    for b in range(_NBUF):
        wait_gather(b)
        scatter(b)

    plsc.subcore_barrier()
    pltpu.sync_copy(acc.at[pl.ds(zb, _ZROWS)], out_hbm.at[c, pl.ds(zb, _ZROWS)])


_edge_agg = functools.partial(
    pl.kernel,
    out_type=jax.ShapeDtypeStruct((_NC, _ACC_ROWS, _D), jnp.float32),
    mesh=plsc.VectorSubcoreMesh(core_axis_name="c", subcore_axis_name="s"),
    scratch_types=[
        [pltpu.VMEM((_CHUNK,), jnp.int32) for _ in range(_NBUF)],
        [pltpu.VMEM((_CHUNK,), jnp.int32) for _ in range(_NBUF)],
        [pltpu.VMEM((_CHUNK, _D), jnp.float32) for _ in range(_NBUF)],
        pltpu.VMEM_SHARED((_ACC_ROWS, _D), jnp.float32),
        [pltpu.SemaphoreType.DMA for _ in range(_NBUF)],
        [pltpu.SemaphoreType.DMA for _ in range(_NBUF)],
    ],
)(_edge_agg_body)


def _mlp_bn_kernel(p_ref, w1_ref, b1_ref, w2_ref, b2_ref, g_ref, be_ref,
                   out_ref):
    pooled = p_ref[0, :_N_NODES, :] + p_ref[1, :_N_NODES, :]
    h = jnp.dot(pooled, w1_ref[...], preferred_element_type=jnp.float32)
    h = jnp.maximum(h + b1_ref[...], 0.0)
    z = jnp.dot(h, w2_ref[...], preferred_element_type=jnp.float32)
    z = z + b2_ref[...]
    mean = jnp.mean(z, axis=0, keepdims=True)
    var = jnp.mean((z - mean) * (z - mean), axis=0, keepdims=True)
    hn = (z - mean) * lax.rsqrt(var + 1e-5) * g_ref[...] + be_ref[...]
    out_ref[...] = jnp.maximum(hn, 0.0)


def _mlp_bn_pool_kernel(p_ref, gid_ref, w1_ref, b1_ref, w2_ref, b2_ref,
                        g_ref, be_ref, out_h_ref, out_p_ref):
    pooled = p_ref[0, :_N_NODES, :] + p_ref[1, :_N_NODES, :]
    h = jnp.dot(pooled, w1_ref[...], preferred_element_type=jnp.float32)
    h = jnp.maximum(h + b1_ref[...], 0.0)
    z = jnp.dot(h, w2_ref[...], preferred_element_type=jnp.float32)
    z = z + b2_ref[...]
    mean = jnp.mean(z, axis=0, keepdims=True)
    var = jnp.mean((z - mean) * (z - mean), axis=0, keepdims=True)
    hn = (z - mean) * lax.rsqrt(var + 1e-5) * g_ref[...] + be_ref[...]
    hr = jnp.maximum(hn, 0.0)
    out_h_ref[...] = hr
    oh = (lax.broadcasted_iota(jnp.int32, (_N_GRAPHS, _N_NODES), 0)
          == gid_ref[...]).astype(jnp.float32)
    out_p_ref[...] = jnp.dot(oh, hr, preferred_element_type=jnp.float32)


def _mlp_bn(p, w1, b1, w2, b2, gamma, beta):
    return pl.pallas_call(
        _mlp_bn_kernel,
        out_shape=jax.ShapeDtypeStruct((_N_NODES, _D), jnp.float32),
    )(p, w1, b1.reshape(1, _D), w2, b2.reshape(1, _D),
      gamma.reshape(1, _D), beta.reshape(1, _D))


def _mlp_bn_pool(p, gids, w1, b1, w2, b2, gamma, beta):
    return pl.pallas_call(
        _mlp_bn_pool_kernel,
        out_shape=(jax.ShapeDtypeStruct((_N_NODES, _D), jnp.float32),
                   jax.ShapeDtypeStruct((_N_GRAPHS, _D), jnp.float32)),
    )(p, gids.reshape(1, _N_NODES), w1, b1.reshape(1, _D), w2,
      b2.reshape(1, _D), gamma.reshape(1, _D), beta.reshape(1, _D))


def kernel(x, edge_index, graph_ids,
           W1_0, b1_0, W2_0, b2_0, gamma_0, beta_0,
           W1_1, b1_1, W2_1, b2_1, gamma_1, beta_1):
    pad = _NW * _EPW - _N_EDGES
    pad_iota = jnp.arange(pad, dtype=jnp.int32)
    src = jnp.concatenate(
        [edge_index[0], pad_iota % _N_NODES]
    ).reshape(_NW, _NCHUNK, _CHUNK)
    n_junk = _ACC_ROWS - _N_NODES
    dst = jnp.concatenate(
        [edge_index[1], _N_NODES + (pad_iota % n_junk)]
    ).reshape(_NW, _NCHUNK, _CHUNK)
    p0 = _edge_agg(x, src, dst)
    h1 = _mlp_bn(p0, W1_0, b1_0, W2_0, b2_0, gamma_0, beta_0)
    p1 = _edge_agg(h1, src, dst)
    h2, pooled_x = _mlp_bn_pool(p1, graph_ids, W1_1, b1_1, W2_1, b2_1,
                                gamma_1, beta_1)
    return (pooled_x, h2)


# speedup vs baseline: 1.0402x; 1.0029x over previous
import functools

import jax
import jax.numpy as jnp
from jax import lax
from jax.experimental import pallas as pl
from jax.experimental.pallas import tpu as pltpu
from jax.experimental.pallas import tpu_sc as plsc

_N_NODES = 10000
_N_EDGES = 320000
_D = 128
_N_GRAPHS = 64

_NC = 2
_NS = 16
_NW = _NC * _NS
_CHUNK = 120
_NCHUNK = 84
_EPW = _NCHUNK * _CHUNK
_NBUF = 2
_ZROWS = 640
_ACC_ROWS = _NS * _ZROWS


def _edge_agg_body(h_hbm, src_hbm, dst_hbm, out_hbm,
                   sidx, didx, rows, acc, isems, gsems):
    c = lax.axis_index("c")
    s = lax.axis_index("s")
    wid = c * _NS + s
    zb = s * _ZROWS

    def issue_idx(g, b):
        pltpu.async_copy(src_hbm.at[wid, g], sidx[b], isems[b])
        pltpu.async_copy(dst_hbm.at[wid, g], didx[b], isems[b])

    def wait_idx(g, b):
        pltpu.make_async_copy(src_hbm.at[wid, g], sidx[b], isems[b]).wait()
        pltpu.make_async_copy(dst_hbm.at[wid, g], didx[b], isems[b]).wait()

    def issue_gather(b):
        pltpu.async_copy(h_hbm.at[sidx[b]], rows[b], gsems[b])

    def wait_gather(b):
        pltpu.make_async_copy(h_hbm.at[sidx[b]], rows[b], gsems[b]).wait()

    def scatter(b):
        pltpu.sync_copy(rows[b], acc.at[didx[b]], add=True)

    for b in range(_NBUF):
        issue_idx(b, b)

    zvec = jnp.zeros((16,), jnp.float32)

    def zstore(i, carry):
        rows[0][i // 8, pl.ds((i % 8) * 16, 16)] = zvec
        return carry

    lax.fori_loop(0, _CHUNK * (_D // 16), zstore, 0)
    for k in range(_ZROWS // _CHUNK):
        pltpu.sync_copy(rows[0], acc.at[pl.ds(zb + k * _CHUNK, _CHUNK)])
    _REM = _ZROWS % _CHUNK
    if _REM:
        pltpu.sync_copy(rows[0].at[pl.ds(0, _REM)],
                        acc.at[pl.ds(zb + (_ZROWS // _CHUNK) * _CHUNK, _REM)])
    plsc.subcore_barrier()

    def super_body(i, carry):
        g0 = i * _NBUF
        for b in range(_NBUF):
            wait_idx(g0 + b, b)
            issue_gather(b)
        for b in range(_NBUF):
            wait_gather(b)
            scatter(b)
            issue_idx(g0 + _NBUF + b, b)
        return carry

    lax.fori_loop(0, _NCHUNK // _NBUF - 1, super_body, 0)

    g0 = _NCHUNK - _NBUF
    for b in range(_NBUF):
        wait_idx(g0 + b, b)
        issue_gather(b)
    for b in range(_NBUF):
        wait_gather(b)
        scatter(b)

    plsc.subcore_barrier()
    pltpu.sync_copy(acc.at[pl.ds(zb, _ZROWS)], out_hbm.at[c, pl.ds(zb, _ZROWS)])


_edge_agg = functools.partial(
    pl.kernel,
    out_type=jax.ShapeDtypeStruct((_NC, _ACC_ROWS, _D), jnp.float32),
    mesh=plsc.VectorSubcoreMesh(core_axis_name="c", subcore_axis_name="s"),
    scratch_types=[
        [pltpu.VMEM((_CHUNK,), jnp.int32) for _ in range(_NBUF)],
        [pltpu.VMEM((_CHUNK,), jnp.int32) for _ in range(_NBUF)],
        [pltpu.VMEM((_CHUNK, _D), jnp.float32) for _ in range(_NBUF)],
        pltpu.VMEM_SHARED((_ACC_ROWS, _D), jnp.float32),
        [pltpu.SemaphoreType.DMA for _ in range(_NBUF)],
        [pltpu.SemaphoreType.DMA for _ in range(_NBUF)],
    ],
)(_edge_agg_body)


def _mlp_bn_kernel(p_ref, w1_ref, b1_ref, w2_ref, b2_ref, g_ref, be_ref,
                   out_ref):
    pooled = p_ref[0, :_N_NODES, :] + p_ref[1, :_N_NODES, :]
    h = jnp.dot(pooled, w1_ref[...], preferred_element_type=jnp.float32)
    h = jnp.maximum(h + b1_ref[...], 0.0)
    z = jnp.dot(h, w2_ref[...], preferred_element_type=jnp.float32)
    z = z + b2_ref[...]
    mean = jnp.mean(z, axis=0, keepdims=True)
    var = jnp.mean((z - mean) * (z - mean), axis=0, keepdims=True)
    hn = (z - mean) * lax.rsqrt(var + 1e-5) * g_ref[...] + be_ref[...]
    out_ref[...] = jnp.maximum(hn, 0.0)


def _mlp_bn_pool_kernel(p_ref, gid_ref, w1_ref, b1_ref, w2_ref, b2_ref,
                        g_ref, be_ref, out_h_ref, out_p_ref):
    pooled = p_ref[0, :_N_NODES, :] + p_ref[1, :_N_NODES, :]
    h = jnp.dot(pooled, w1_ref[...], preferred_element_type=jnp.float32)
    h = jnp.maximum(h + b1_ref[...], 0.0)
    z = jnp.dot(h, w2_ref[...], preferred_element_type=jnp.float32)
    z = z + b2_ref[...]
    mean = jnp.mean(z, axis=0, keepdims=True)
    var = jnp.mean((z - mean) * (z - mean), axis=0, keepdims=True)
    hn = (z - mean) * lax.rsqrt(var + 1e-5) * g_ref[...] + be_ref[...]
    hr = jnp.maximum(hn, 0.0)
    out_h_ref[...] = hr
    oh = (lax.broadcasted_iota(jnp.int32, (_N_GRAPHS, _N_NODES), 0)
          == gid_ref[...]).astype(jnp.float32)
    out_p_ref[...] = jnp.dot(oh, hr, preferred_element_type=jnp.float32)


def _mlp_bn(p, w1, b1, w2, b2, gamma, beta):
    return pl.pallas_call(
        _mlp_bn_kernel,
        out_shape=jax.ShapeDtypeStruct((_N_NODES, _D), jnp.float32),
    )(p, w1, b1.reshape(1, _D), w2, b2.reshape(1, _D),
      gamma.reshape(1, _D), beta.reshape(1, _D))


def _mlp_bn_pool(p, gids, w1, b1, w2, b2, gamma, beta):
    return pl.pallas_call(
        _mlp_bn_pool_kernel,
        out_shape=(jax.ShapeDtypeStruct((_N_NODES, _D), jnp.float32),
                   jax.ShapeDtypeStruct((_N_GRAPHS, _D), jnp.float32)),
    )(p, gids.reshape(1, _N_NODES), w1, b1.reshape(1, _D), w2,
      b2.reshape(1, _D), gamma.reshape(1, _D), beta.reshape(1, _D))


def kernel(x, edge_index, graph_ids,
           W1_0, b1_0, W2_0, b2_0, gamma_0, beta_0,
           W1_1, b1_1, W2_1, b2_1, gamma_1, beta_1):
    pad = _NW * _EPW - _N_EDGES
    pad_iota = jnp.arange(pad, dtype=jnp.int32)
    src = jnp.concatenate(
        [edge_index[0], pad_iota % _N_NODES]
    ).reshape(_NW, _NCHUNK, _CHUNK)
    n_junk = _ACC_ROWS - _N_NODES
    dst = jnp.concatenate(
        [edge_index[1], _N_NODES + (pad_iota % n_junk)]
    ).reshape(_NW, _NCHUNK, _CHUNK)
    p0 = _edge_agg(x, src, dst)
    h1 = _mlp_bn(p0, W1_0, b1_0, W2_0, b2_0, gamma_0, beta_0)
    p1 = _edge_agg(h1, src, dst)
    h2, pooled_x = _mlp_bn_pool(p1, graph_ids, W1_1, b1_1, W2_1, b2_1,
                                gamma_1, beta_1)
    return (pooled_x, h2)
